# baseline (speedup 1.0000x reference)
import functools

import jax
import jax.numpy as jnp
from jax import lax
from jax.experimental import pallas as pl
from jax.experimental.pallas import tpu as pltpu
from jax.experimental.pallas import tpu_sc as plsc

_WH = 48
_NBUF = 4


def _window_metadata(I, H, W, NCK, CHW, NJ, NJP):
    Ic = I.reshape(NCK, CHW)
    first = Ic[:, ::16]
    last = Ic[:, 15::16]
    WSEG = _WH * W
    v_lo = Ic[:, 0] // WSEG
    v_hi = Ic[:, -1] // WSEG
    nw = (v_hi - v_lo + 1).astype(jnp.int32)
    h0 = (v_lo * _WH).astype(jnp.int32)
    j = jnp.arange(NJ, dtype=jnp.int32)
    b = (v_lo[:, None] + j[None, :]) * WSEG

    def count_lt(arr, q):
        return jnp.sum(
            arr[:, None, :] < q[:, :, None], axis=-1, dtype=jnp.int32
        )

    gs = count_lt(last, b)
    gsi = count_lt(first, b)
    gei = count_lt(last, b + WSEG)
    ge = count_lt(first, b + WSEG)
    valid = j[None, :] < nw[:, None]
    z = jnp.zeros_like(gs)
    lanes = [
        jnp.where(valid, gs, 0),
        jnp.where(valid, gsi, 0),
        jnp.where(valid, gei, 0),
        jnp.where(valid, ge, 0),
        jnp.broadcast_to(nw[:, None], gs.shape),
        jnp.broadcast_to(h0[:, None], gs.shape),
    ] + [z] * 10
    meta = jnp.stack(lanes, axis=-1)
    meta = jnp.concatenate(
        [meta, jnp.zeros((NCK, NJP - NJ, 16), jnp.int32)], axis=1
    )
    return meta.reshape(-1)


def kernel(x, I):
    B, C, H, W = x.shape
    HW = H * W
    R = B * C
    M = I.shape[0]

    NC, NS = 2, 16
    NRG = 4
    NCK = 8
    RG = R // NRG
    CHW = M // NCK
    NG = CHW // 16
    NJ = H // _WH
    NJP = ((NJ + 15) // 16) * 16
    MROW = NJP * 16
    assert RG * NRG == R and CHW * NCK == M and NG * 16 == CHW

    meta = _window_metadata(I, H, W, NCK, CHW, NJ, NJP)
    Ih = (I // W).astype(jnp.int32)
    Iw = (I % W).astype(jnp.int32)

    mesh = plsc.VectorSubcoreMesh(core_axis_name="c", subcore_axis_name="s")

    @functools.partial(
        pl.kernel,
        mesh=mesh,
        compiler_params=pltpu.CompilerParams(needs_layout_passes=False),
        out_type=jax.ShapeDtypeStruct((R * M,), jnp.float32),
        scratch_types=[
            pltpu.VMEM((CHW,), jnp.int32),
            pltpu.VMEM((CHW,), jnp.int32),
            pltpu.VMEM((_NBUF * _WH, W), jnp.float32),
            pltpu.VMEM((2 * CHW,), jnp.float32),
            pltpu.VMEM((MROW,), jnp.int32),
            pltpu.SemaphoreType.DMA,
            pltpu.SemaphoreType.DMA,
        ],
    )
    def k(x_hbm, ih_hbm, iw_hbm, meta_hbm, out_hbm, ih_ref, iw_ref, win,
          outbuf, meta_v, wsem, osem):
        cid = lax.axis_index("c")
        sid = lax.axis_index("s")
        wid = sid * NC + cid
        rg = wid // NCK
        ck = lax.rem(wid, NCK)
        rbase = rg * RG

        pltpu.sync_copy(
            ih_hbm.at[pl.ds(pl.multiple_of(ck * CHW, 8), CHW)], ih_ref
        )
        pltpu.sync_copy(
            iw_hbm.at[pl.ds(pl.multiple_of(ck * CHW, 8), CHW)], iw_ref
        )
        pltpu.sync_copy(
            meta_hbm.at[pl.ds(pl.multiple_of(ck * MROW, 8), MROW)], meta_v
        )
        m0 = meta_v[pl.ds(0, 16)]
        nw = m0[4]
        h0 = m0[5]
        total = nw * RG

        def out_off(row):
            return pl.ds(pl.multiple_of((row * NCK + ck) * CHW, 8), CHW)

        def win_src(row, j):
            hj = h0 + j * jnp.int32(_WH)
            return x_hbm.at[row, pl.ds(pl.multiple_of(hj, 8), _WH), :]

        def win_dst(sl):
            return win.at[
                pl.ds(pl.multiple_of(sl * jnp.int32(_WH), 8), _WH), :
            ]

        def bump(row, j):
            wrap = j + 1 >= nw
            return (
                jnp.where(wrap, row + 1, row),
                jnp.where(wrap, 0, j + 1),
            )

        rp, jp = jnp.int32(rbase), jnp.int32(0)
        for s in range(_NBUF - 1):
            @pl.when(s < RG)
            def _(rp=rp, jp=jp, s=s):
                pltpu.async_copy(win_src(rp, jp), win_dst(jnp.int32(s)), wsem)
            rp, jp = bump(rp, jp)

        def task_body(t, carry):
            row, j, rp, jp = carry
            sl = lax.rem(t, _NBUF)

            pltpu.make_async_copy(win_src(row, j), win_dst(sl), wsem).wait()

            @pl.when(t + (_NBUF - 1) < total)
            def _():
                pltpu.async_copy(
                    win_src(rp, jp),
                    win_dst(lax.rem(t + (_NBUF - 1), _NBUF)),
                    wsem,
                )

            mrow = meta_v[pl.ds(j * 16, 16)]
            gs = mrow[0]
            gsi = mrow[1]
            gei = mrow[2]
            ge = mrow[3]
            hj = h0 + j * jnp.int32(_WH)
            hb = hj - sl * jnp.int32(_WH)
            oo = lax.rem(row, 2) * jnp.int32(CHW)

            @pl.when((j == 0) & (row >= rbase + 2))
            def _():
                pltpu.make_async_copy(
                    outbuf.at[pl.ds(pl.multiple_of(oo, 8), CHW)],
                    out_hbm.at[out_off(row - 2)],
                    osem,
                ).wait()

            def masked_group(g):
                hv = ih_ref[pl.ds(g * 16, 16)]
                wv = iw_ref[pl.ds(g * 16, 16)]
                m = (hv >= hj) & (hv < hj + _WH)
                dh2 = jnp.minimum(
                    jnp.maximum(hv - hb, sl * jnp.int32(_WH)),
                    sl * jnp.int32(_WH) + jnp.int32(_WH - 1),
                )
                vals = plsc.load_gather(win, [dh2, wv], mask=m)
                prev = outbuf[pl.ds(oo + g * 16, 16)]
                outbuf[pl.ds(oo + g * 16, 16)] = jnp.where(m, vals, prev)

            @pl.when(gs < gsi)
            def _():
                masked_group(gs)

            @pl.when(gei < ge)
            def _():
                masked_group(gei)

            gsi2 = jnp.minimum(gsi, gei)

            @plsc.parallel_loop(gsi2, gei, unroll=8)
            def _(g):
                hv = ih_ref[pl.ds(g * 16, 16)]
                wv = iw_ref[pl.ds(g * 16, 16)]
                vals = plsc.load_gather(win, [hv - hb, wv])
                outbuf[pl.ds(oo + g * 16, 16)] = vals

            @pl.when(j + 1 >= nw)
            def _():
                pltpu.async_copy(
                    outbuf.at[pl.ds(pl.multiple_of(oo, 8), CHW)],
                    out_hbm.at[out_off(row)],
                    osem,
                )

            row, j = bump(row, j)
            rp, jp = bump(rp, jp)
            return (row, j, rp, jp)

        lax.fori_loop(
            0,
            total,
            task_body,
            (jnp.int32(rbase), jnp.int32(0), rp, jp),
            unroll=False,
        )

        for back in (2, 1):
            oo = ((RG - back) % 2) * CHW
            pltpu.make_async_copy(
                outbuf.at[pl.ds(oo, CHW)],
                out_hbm.at[out_off(rbase + RG - back)],
                osem,
            ).wait()

    out = k(x.reshape(R, H, W), Ih, Iw, meta)
    return out.reshape(B, C, M)


# speedup vs baseline: 1.1958x; 1.1958x over previous
import functools

import jax
import jax.numpy as jnp
from jax import lax
from jax.experimental import pallas as pl
from jax.experimental.pallas import tpu as pltpu
from jax.experimental.pallas import tpu_sc as plsc

_WH = 32
_NBUF = 6


def _window_metadata(I, H, W, NCK, CHW, NJ, NJP):
    Ic = I.reshape(NCK, CHW)
    first = Ic[:, ::16]
    last = Ic[:, 15::16]
    WSEG = _WH * W
    v_lo = Ic[:, 0] // WSEG
    v_hi = Ic[:, -1] // WSEG
    nw = (v_hi - v_lo + 1).astype(jnp.int32)
    h0 = (v_lo * _WH).astype(jnp.int32)
    j = jnp.arange(NJ, dtype=jnp.int32)
    b = (v_lo[:, None] + j[None, :]) * WSEG

    def count_lt(arr, q):
        return jnp.sum(
            arr[:, None, :] < q[:, :, None], axis=-1, dtype=jnp.int32
        )

    gs = count_lt(last, b)
    gsi = count_lt(first, b)
    gei = count_lt(last, b + WSEG)
    ge = count_lt(first, b + WSEG)
    valid = j[None, :] < nw[:, None]
    z = jnp.zeros_like(gs)
    lanes = [
        jnp.where(valid, gs, 0),
        jnp.where(valid, gsi, 0),
        jnp.where(valid, gei, 0),
        jnp.where(valid, ge, 0),
        jnp.broadcast_to(nw[:, None], gs.shape),
        jnp.broadcast_to(h0[:, None], gs.shape),
    ] + [z] * 10
    meta = jnp.stack(lanes, axis=-1)
    meta = jnp.concatenate(
        [meta, jnp.zeros((NCK, NJP - NJ, 16), jnp.int32)], axis=1
    )
    return meta.reshape(-1)


def kernel(x, I):
    B, C, H, W = x.shape
    HW = H * W
    R = B * C
    M = I.shape[0]

    NC, NS = 2, 16
    NRG = 4
    NCK = 8
    RG = R // NRG
    CHW = M // NCK
    NG = CHW // 16
    NJ = H // _WH
    NJP = ((NJ + 15) // 16) * 16
    MROW = NJP * 16
    assert RG * NRG == R and CHW * NCK == M and NG * 16 == CHW

    meta = _window_metadata(I, H, W, NCK, CHW, NJ, NJP)
    Ih = (I // W).astype(jnp.int32)
    Iw = (I % W).astype(jnp.int32)

    mesh = plsc.VectorSubcoreMesh(core_axis_name="c", subcore_axis_name="s")

    @functools.partial(
        pl.kernel,
        mesh=mesh,
        compiler_params=pltpu.CompilerParams(needs_layout_passes=False),
        out_type=jax.ShapeDtypeStruct((R * M,), jnp.float32),
        scratch_types=[
            pltpu.VMEM((CHW,), jnp.int32),
            pltpu.VMEM((CHW,), jnp.int32),
            pltpu.VMEM((_NBUF * _WH, W), jnp.float32),
            pltpu.VMEM((2 * CHW,), jnp.float32),
            pltpu.VMEM((MROW,), jnp.int32),
            pltpu.SemaphoreType.DMA,
            pltpu.SemaphoreType.DMA,
        ],
    )
    def k(x_hbm, ih_hbm, iw_hbm, meta_hbm, out_hbm, ih_ref, iw_ref, win,
          outbuf, meta_v, wsem, osem):
        cid = lax.axis_index("c")
        sid = lax.axis_index("s")
        wid = sid * NC + cid
        rg = wid // NCK
        ck = lax.rem(wid, NCK)
        rbase = rg * RG

        pltpu.sync_copy(
            ih_hbm.at[pl.ds(pl.multiple_of(ck * CHW, 8), CHW)], ih_ref
        )
        pltpu.sync_copy(
            iw_hbm.at[pl.ds(pl.multiple_of(ck * CHW, 8), CHW)], iw_ref
        )
        pltpu.sync_copy(
            meta_hbm.at[pl.ds(pl.multiple_of(ck * MROW, 8), MROW)], meta_v
        )
        m0 = meta_v[pl.ds(0, 16)]
        nw = m0[4]
        h0 = m0[5]
        total = nw * RG

        def out_off(row):
            return pl.ds(pl.multiple_of((row * NCK + ck) * CHW, 8), CHW)

        def win_src(row, j):
            hj = h0 + j * jnp.int32(_WH)
            return x_hbm.at[row, pl.ds(pl.multiple_of(hj, 8), _WH), :]

        def win_dst(sl):
            return win.at[
                pl.ds(pl.multiple_of(sl * jnp.int32(_WH), 8), _WH), :
            ]

        def bump(row, j):
            wrap = j + 1 >= nw
            return (
                jnp.where(wrap, row + 1, row),
                jnp.where(wrap, 0, j + 1),
            )

        rp, jp = jnp.int32(rbase), jnp.int32(0)
        for s in range(_NBUF - 1):
            @pl.when(s < RG)
            def _(rp=rp, jp=jp, s=s):
                pltpu.async_copy(win_src(rp, jp), win_dst(jnp.int32(s)), wsem)
            rp, jp = bump(rp, jp)

        def task_body(t, carry):
            row, j, rp, jp = carry
            sl = lax.rem(t, _NBUF)

            pltpu.make_async_copy(win_src(row, j), win_dst(sl), wsem).wait()

            @pl.when(t + (_NBUF - 1) < total)
            def _():
                pltpu.async_copy(
                    win_src(rp, jp),
                    win_dst(lax.rem(t + (_NBUF - 1), _NBUF)),
                    wsem,
                )

            mrow = meta_v[pl.ds(j * 16, 16)]
            gs = mrow[0]
            gsi = mrow[1]
            gei = mrow[2]
            ge = mrow[3]
            hj = h0 + j * jnp.int32(_WH)
            hb = hj - sl * jnp.int32(_WH)
            oo = lax.rem(row, 2) * jnp.int32(CHW)

            @pl.when((j == 0) & (row >= rbase + 2))
            def _():
                pltpu.make_async_copy(
                    outbuf.at[pl.ds(pl.multiple_of(oo, 8), CHW)],
                    out_hbm.at[out_off(row - 2)],
                    osem,
                ).wait()

            def masked_group(g):
                hv = ih_ref[pl.ds(g * 16, 16)]
                wv = iw_ref[pl.ds(g * 16, 16)]
                m = (hv >= hj) & (hv < hj + _WH)
                dh2 = jnp.minimum(
                    jnp.maximum(hv - hb, sl * jnp.int32(_WH)),
                    sl * jnp.int32(_WH) + jnp.int32(_WH - 1),
                )
                vals = plsc.load_gather(win, [dh2, wv], mask=m)
                prev = outbuf[pl.ds(oo + g * 16, 16)]
                outbuf[pl.ds(oo + g * 16, 16)] = jnp.where(m, vals, prev)

            @pl.when(gs < gsi)
            def _():
                masked_group(gs)

            @pl.when(gei < ge)
            def _():
                masked_group(gei)

            gsi2 = jnp.minimum(gsi, gei)

            @plsc.parallel_loop(gsi2, gei, unroll=8)
            def _(g):
                hv = ih_ref[pl.ds(g * 16, 16)]
                wv = iw_ref[pl.ds(g * 16, 16)]
                vals = plsc.load_gather(win, [hv - hb, wv])
                outbuf[pl.ds(oo + g * 16, 16)] = vals

            @pl.when(j + 1 >= nw)
            def _():
                pltpu.async_copy(
                    outbuf.at[pl.ds(pl.multiple_of(oo, 8), CHW)],
                    out_hbm.at[out_off(row)],
                    osem,
                )

            row, j = bump(row, j)
            rp, jp = bump(rp, jp)
            return (row, j, rp, jp)

        lax.fori_loop(
            0,
            total,
            task_body,
            (jnp.int32(rbase), jnp.int32(0), rp, jp),
            unroll=False,
        )

        for back in (2, 1):
            oo = ((RG - back) % 2) * CHW
            pltpu.make_async_copy(
                outbuf.at[pl.ds(oo, CHW)],
                out_hbm.at[out_off(rbase + RG - back)],
                osem,
            ).wait()

    out = k(x.reshape(R, H, W), Ih, Iw, meta)
    return out.reshape(B, C, M)
